# baseline (speedup 1.0000x reference)
import functools

import jax
import jax.numpy as jnp
from jax import lax
from jax.experimental import pallas as pl
from jax.experimental.pallas import tpu as pltpu
from jax.experimental.pallas import tpu_sc as plsc

_D = 50
_V = 100
_LANES = 128
_NC = 2
_NS = 16
_NW = _NC * _NS
_CR = 8


@functools.cache
def _build(rows):
    rows_per_w = rows // _NW
    nchunks = rows_per_w // _CR
    mesh = plsc.VectorSubcoreMesh(
        core_axis_name="c", subcore_axis_name="s",
        num_cores=_NC, num_subcores=_NS,
    )

    @functools.partial(
        pl.kernel,
        out_type=jax.ShapeDtypeStruct((_D, rows, _LANES), jnp.float32),
        mesh=mesh,
        scratch_types=[
            pltpu.VMEM((_D * _LANES,), jnp.float32),
            pltpu.VMEM((2, _CR, _LANES), jnp.int32),
            pltpu.VMEM((2, _D, _CR, _LANES), jnp.float32),
            pltpu.SemaphoreType.DMA((2,)),
            pltpu.SemaphoreType.DMA((2,)),
            pltpu.SemaphoreType.DMA,
        ],
        compiler_params=pltpu.CompilerParams(
            needs_layout_passes=False,
            disable_bounds_checks=True,
        ),
    )
    def emb(idx_hbm, table_hbm, out_hbm, table_v, idx_v, out_v,
            sem_i, sem_o, sem_t):
        wid = lax.axis_index("s") * _NC + lax.axis_index("c")
        row0 = wid * rows_per_w
        pltpu.async_copy(table_hbm, table_v, sem_t).wait()
        pltpu.async_copy(idx_hbm.at[pl.ds(row0, _CR)], idx_v.at[0],
                         sem_i.at[0])
        pltpu.async_copy(idx_hbm.at[pl.ds(row0 + _CR, _CR)], idx_v.at[1],
                         sem_i.at[1])

        def chunk_body(c, carry):
            s = c % 2
            r0 = row0 + c * _CR
            pltpu.make_async_copy(
                idx_hbm.at[pl.ds(row0, _CR)], idx_v.at[s], sem_i.at[s]
            ).wait()

            @pl.when(c >= 2)
            def _():
                pltpu.make_async_copy(
                    out_v.at[s], out_hbm.at[:, pl.ds(row0, _CR), :],
                    sem_o.at[s],
                ).wait()

            @plsc.parallel_loop(0, _CR, unroll=2)
            def _row_body(r):
                for l in range(_LANES // 16):
                    idx_vec = idx_v[s, r, pl.ds(l * 16, 16)]
                    for d in range(_D):
                        vals = plsc.load_gather(
                            table_v, [idx_vec + (d * _LANES)])
                        out_v[s, d, r, pl.ds(l * 16, 16)] = vals

            pltpu.async_copy(out_v.at[s], out_hbm.at[:, pl.ds(r0, _CR), :],
                             sem_o.at[s])
            cn = jnp.minimum(c + 2, nchunks - 1)
            pltpu.async_copy(
                idx_hbm.at[pl.ds(row0 + cn * _CR, _CR)], idx_v.at[s],
                sem_i.at[s])
            return carry

        lax.fori_loop(0, nchunks, chunk_body, 0)

        for s in range(2):
            pltpu.make_async_copy(
                out_v.at[s], out_hbm.at[:, pl.ds(row0, _CR), :], sem_o.at[s]
            ).wait()
            pltpu.make_async_copy(
                idx_hbm.at[pl.ds(row0, _CR)], idx_v.at[s], sem_i.at[s]
            ).wait()

    return emb


def kernel(indices, table):
    b, t = indices.shape
    rows = (b * t) // _LANES
    idx_t = indices.T.reshape(rows, _LANES)
    table_t = jnp.pad(table.T, ((0, 0), (0, _LANES - table.shape[0])))
    out3 = _build(rows)(idx_t, table_t.reshape(-1))
    return out3.reshape(_D, t, b).transpose(2, 1, 0)


# speedup vs baseline: 1.9782x; 1.9782x over previous
import functools

import jax
import jax.numpy as jnp
from jax import lax
from jax.experimental import pallas as pl
from jax.experimental.pallas import tpu as pltpu
from jax.experimental.pallas import tpu_sc as plsc

_D = 50
_V = 100
_LANES = 128
_NC = 2
_NS = 16
_NW = _NC * _NS
_CR = 8


@functools.cache
def _build(rows):
    rows_per_w = rows // _NW
    nchunks = rows_per_w // _CR
    mesh = plsc.VectorSubcoreMesh(
        core_axis_name="c", subcore_axis_name="s",
        num_cores=_NC, num_subcores=_NS,
    )

    @functools.partial(
        pl.kernel,
        out_type=jax.ShapeDtypeStruct((_D, rows, _LANES), jnp.float32),
        mesh=mesh,
        scratch_types=[
            pltpu.VMEM((_D * _LANES,), jnp.float32),
            pltpu.VMEM((2, _CR, _LANES), jnp.int32),
            pltpu.VMEM((2, _D, _CR, _LANES), jnp.float32),
            pltpu.SemaphoreType.DMA((2,)),
            pltpu.SemaphoreType.DMA((2,)),
            pltpu.SemaphoreType.DMA,
        ],
        compiler_params=pltpu.CompilerParams(
            needs_layout_passes=False,
            disable_bounds_checks=True,
        ),
    )
    def emb(idx_hbm, table_hbm, out_hbm, table_v, idx_v, out_v,
            sem_i, sem_o, sem_t):
        wid = lax.axis_index("s") * _NC + lax.axis_index("c")
        row0 = wid * rows_per_w
        pltpu.async_copy(table_hbm, table_v, sem_t).wait()
        pltpu.async_copy(idx_hbm.at[pl.ds(row0, _CR)], idx_v.at[0],
                         sem_i.at[0])
        pltpu.async_copy(idx_hbm.at[pl.ds(row0 + _CR, _CR)], idx_v.at[1],
                         sem_i.at[1])

        def chunk_body(c, carry):
            s = c % 2
            r0 = row0 + c * _CR
            pltpu.make_async_copy(
                idx_hbm.at[pl.ds(row0, _CR)], idx_v.at[s], sem_i.at[s]
            ).wait()

            @pl.when(c >= 2)
            def _():
                pltpu.make_async_copy(
                    out_v.at[s], out_hbm.at[:, pl.ds(row0, _CR), :],
                    sem_o.at[s],
                ).wait()

            def row_body(r, carry2):
                for l in range(_LANES // 16):
                    idx_vec = idx_v[s, r, pl.ds(l * 16, 16)]
                    for d0 in range(0, _D, 10):
                        vals = [
                            plsc.load_gather(
                                table_v, [idx_vec + ((d0 + k) * _LANES)])
                            for k in range(10)
                        ]
                        for k in range(10):
                            out_v[s, d0 + k, r, pl.ds(l * 16, 16)] = vals[k]
                return carry2

            lax.fori_loop(0, _CR, row_body, 0)

            pltpu.async_copy(out_v.at[s], out_hbm.at[:, pl.ds(r0, _CR), :],
                             sem_o.at[s])
            cn = jnp.minimum(c + 2, nchunks - 1)
            pltpu.async_copy(
                idx_hbm.at[pl.ds(row0 + cn * _CR, _CR)], idx_v.at[s],
                sem_i.at[s])
            return carry

        lax.fori_loop(0, nchunks, chunk_body, 0)

        for s in range(2):
            pltpu.make_async_copy(
                out_v.at[s], out_hbm.at[:, pl.ds(row0, _CR), :], sem_o.at[s]
            ).wait()
            pltpu.make_async_copy(
                idx_hbm.at[pl.ds(row0, _CR)], idx_v.at[s], sem_i.at[s]
            ).wait()

    return emb


def kernel(indices, table):
    b, t = indices.shape
    rows = (b * t) // _LANES
    idx_t = indices.T.reshape(rows, _LANES)
    table_t = jnp.pad(table.T, ((0, 0), (0, _LANES - table.shape[0])))
    out3 = _build(rows)(idx_t, table_t.reshape(-1))
    return out3.reshape(_D, t, b).transpose(2, 1, 0)
